# baseline (speedup 1.0000x reference)
import functools

import numpy as np
import jax
import jax.numpy as jnp
from jax import lax
from jax.experimental import pallas as pl
from jax.experimental.pallas import tpu as pltpu
from jax.experimental.pallas import tpu_sc as plsc


def _pos_encoding_np(length: int, depth: int) -> np.ndarray:
    half = depth / 2
    positions = np.arange(length)[:, np.newaxis]
    depths = np.arange(half)[np.newaxis, :] / half
    angle_rates = 1 / 10000 ** depths
    angle_rads = positions * angle_rates
    return np.concatenate(
        [np.sin(angle_rads), np.cos(angle_rads)], axis=-1
    ).astype(np.float32)


def kernel(x, table):
    B, L = x.shape
    V, D = table.shape
    N = B * L
    scale = float(np.sqrt(float(D)))

    info = plsc.get_sparse_core_info()
    NC, NS, LN = info.num_cores, info.num_subcores, info.num_lanes
    NW = NC * NS
    P = L // NW
    VPR = D // LN
    SUB = 4
    PS = P // SUB

    pos = jnp.asarray(_pos_encoding_np(L, D))
    xt = jnp.transpose(x).reshape(NW * SUB, PS * B)

    mesh = plsc.VectorSubcoreMesh(core_axis_name="c", subcore_axis_name="s")

    @functools.partial(
        pl.kernel,
        mesh=mesh,
        out_type=jax.ShapeDtypeStruct((N, D), jnp.float32),
        scratch_types=[
            pltpu.VMEM((SUB, PS * B), jnp.int32),
            pltpu.VMEM((P * B, D), jnp.float32),
            pltpu.VMEM((P * B, D), jnp.float32),
            pltpu.VMEM((P, D), jnp.float32),
            pltpu.SemaphoreType.DMA,
            pltpu.SemaphoreType.DMA,
            pltpu.SemaphoreType.DMA,
        ],
    )
    def emb_kernel(
        x_hbm, tab_hbm, pos_hbm, out_hbm, idx_v, rows_v, out_v, pos_v, gsem, osem, isem
    ):
        wid = lax.axis_index("s") * NC + lax.axis_index("c")
        pltpu.async_copy(
            x_hbm.at[pl.ds(wid * SUB, SUB)], idx_v, isem
        ).wait()
        gathers = [
            pltpu.async_copy(
                tab_hbm.at[idx_v.at[j]],
                rows_v.at[pl.ds(j * PS * B, PS * B)],
                gsem,
            )
            for j in range(SUB)
        ]
        pltpu.sync_copy(pos_hbm.at[pl.ds(wid * P, P)], pos_v)

        wb = []
        for j in range(SUB):
            gathers[j].wait()

            def body(i, carry, j=j):
                p = j * PS + i
                pvecs = [pos_v[p, pl.ds(k * LN, LN)] for k in range(VPR)]
                for b in range(B):
                    r = (j * PS + i) * B + b
                    o = b * P + p
                    for k in range(VPR):
                        sl = pl.ds(k * LN, LN)
                        out_v[o, sl] = rows_v[r, sl] * scale + pvecs[k]
                return carry

            lax.fori_loop(0, PS, body, 0)
            for b in range(B):
                wb.append(
                    pltpu.async_copy(
                        out_v.at[pl.ds(b * P + j * PS, PS)],
                        out_hbm.at[pl.ds(b * L + wid * P + j * PS, PS)],
                        osem,
                    )
                )
        for c in wb:
            c.wait()

    out = emb_kernel(xt, table, pos)
    return out.reshape(B, L, D)
